# Initial kernel scaffold; baseline (speedup 1.0000x reference)
import jax
import jax.numpy as jnp
from jax.experimental import pallas as pl


def kernel(x_in, adj, idx, W1, a1, W2, a2, W3, a3, fc_W, fc_b, bn_gamma, bn_beta):
    raise NotImplementedError("write your pallas kernel here")


# speedup vs baseline: 14.7668x; 14.7668x over previous
import functools

import jax
import jax.numpy as jnp
from jax import lax
from jax.experimental import pallas as pl
from jax.experimental.pallas import tpu as pltpu
from jax.experimental.pallas import tpu_sc as plsc

N = 10000
E = 320000
D = 128
G = 256
C = 16
WPAD = 144
ALPHA = 0.05
BN_EPS = 1e-5

NC, NS = 2, 16
NTILE = NC * NS
CHUNK = 40
CPT = E // (NTILE * CHUNK)
RB = 3
RI = 6
LANES = 16

BLK = 400
NBLK = N // BLK


def _leaky(v):
    return jnp.where(v >= 0, v, ALPHA * v)


def _tc_layer_body(is_first, x_ref, W_ref, z_ref):
    if is_first:
        x = x_ref[...]
    else:
        p = x_ref[0] + x_ref[1]
        den = p[:, D:D + 1]
        den = jnp.where(den > 0.0, den, 1.0)
        x = _leaky(p[:, :D] / den)
    z = lax.dot_general(x, W_ref[...], (((1,), (1,)), ((), ())),
                        preferred_element_type=jnp.float32)
    z_ref[:, :D] = z
    col = lax.broadcasted_iota(jnp.int32, (BLK, WPAD - D), 1)
    z_ref[:, D:] = jnp.where(col == 0, 1.0, 0.0)


def _tc_layer(x_or_p, W, a, is_first):
    if is_first:
        x_spec = pl.BlockSpec((BLK, D), lambda i: (i, 0))
    else:
        x_spec = pl.BlockSpec((2, BLK, WPAD), lambda i: (0, i, 0))
    zpad = pl.pallas_call(
        functools.partial(_tc_layer_body, is_first),
        grid=(NBLK,),
        in_specs=[
            x_spec,
            pl.BlockSpec((D, D), lambda i: (0, 0)),
        ],
        out_specs=pl.BlockSpec((BLK, WPAD), lambda i: (i, 0)),
        out_shape=jax.ShapeDtypeStruct((N, WPAD), jnp.float32),
    )(x_or_p, W)
    s1, s2 = pl.pallas_call(
        _tc_attn_body,
        in_specs=[
            pl.BlockSpec((N, WPAD), lambda: (0, 0)),
            pl.BlockSpec((1, 2 * D), lambda: (0, 0)),
        ],
        out_specs=[
            pl.BlockSpec((N,), lambda: (0,)),
            pl.BlockSpec((N,), lambda: (0,)),
        ],
        out_shape=[
            jax.ShapeDtypeStruct((N,), jnp.float32),
            jax.ShapeDtypeStruct((N,), jnp.float32),
        ],
    )(zpad, a)
    return zpad, s1, s2


def _tc_attn_body(z_ref, a_ref, s1_ref, s2_ref):
    z = z_ref[:, :D]
    s1_ref[...] = jnp.sum(z * a_ref[0:1, :D], axis=1)
    s2_ref[...] = jnp.sum(z * a_ref[0:1, D:], axis=1)


def _sc_edge(zpad, s1, s2, eidx):
    mesh = plsc.VectorSubcoreMesh(core_axis_name="c", subcore_axis_name="s",
                                  num_cores=NC, num_subcores=NS)
    nstripe = N // NS
    zrows = 25

    @functools.partial(
        pl.kernel,
        out_type=jax.ShapeDtypeStruct((NC, NS, N // NS, WPAD), jnp.float32),
        mesh=mesh,
        compiler_params=pltpu.CompilerParams(use_tc_tiling_on_sc=False,
                                             needs_layout_passes=False),
        scratch_types=[
            pltpu.VMEM((N,), jnp.float32),
            pltpu.VMEM((N,), jnp.float32),
            pltpu.VMEM((RI, 2, CHUNK), jnp.int32),
            pltpu.VMEM((RB, CHUNK, WPAD), jnp.float32),
            pltpu.VMEM_SHARED((N, WPAD), jnp.float32),
            pltpu.SemaphoreType.DMA((RB,)),
            pltpu.SemaphoreType.DMA((RB,)),
            pltpu.SemaphoreType.DMA((RI,)),
        ],
    )
    def k(zpad_hbm, s1_hbm, s2_hbm, eidx_hbm, out_hbm,
          s1_v, s2_v, idx_v, rows_v, acc, gsem, ssem, isem):
        core = lax.axis_index("c")
        sid = lax.axis_index("s")
        w = core * NS + sid

        def issue_idx(c):
            q = lax.rem(c, RI)
            pltpu.async_copy(eidx_hbm.at[w].at[c], idx_v.at[q], isem.at[q])

        def wait_idx(c):
            q = lax.rem(c, RI)
            pltpu.make_async_copy(eidx_hbm.at[w].at[c], idx_v.at[q],
                                  isem.at[q]).wait()

        def issue_gather(c):
            p = lax.rem(c, RB)
            q = lax.rem(c, RI)
            pltpu.async_copy(zpad_hbm.at[idx_v.at[q].at[0]], rows_v.at[p],
                             gsem.at[p])

        def wait_gather(c):
            p = lax.rem(c, RB)
            q = lax.rem(c, RI)
            pltpu.make_async_copy(zpad_hbm.at[idx_v.at[q].at[0]],
                                  rows_v.at[p], gsem.at[p]).wait()

        def issue_scatter(c):
            p = lax.rem(c, RB)
            q = lax.rem(c, RI)
            pltpu.async_copy(rows_v.at[p], acc.at[idx_v.at[q].at[1]],
                             ssem.at[p], add=True)

        def wait_scatter(c):
            p = lax.rem(c, RB)
            q = lax.rem(c, RI)
            pltpu.make_async_copy(rows_v.at[p], acc.at[idx_v.at[q].at[1]],
                                  ssem.at[p]).wait()

        pltpu.sync_copy(s1_hbm, s1_v)
        pltpu.sync_copy(s2_hbm, s2_v)
        for c in range(3):
            issue_idx(c)

        zv = jnp.zeros((LANES,), jnp.float32)

        def zrow_body(i, carry):
            for g in range(WPAD // LANES):
                rows_v[0, i, pl.ds(g * LANES, LANES)] = zv
            return carry

        lax.fori_loop(0, zrows, zrow_body, 0)

        def zcp_body(t, carry):
            pltpu.sync_copy(
                rows_v.at[0].at[pl.ds(0, zrows)],
                acc.at[pl.ds(sid * nstripe + t * zrows, zrows)])
            return carry

        lax.fori_loop(0, nstripe // zrows, zcp_body, 0)
        plsc.subcore_barrier()

        wait_idx(0)
        issue_gather(0)
        wait_idx(1)
        issue_gather(1)
        issue_idx(3)
        issue_idx(4)

        def work(c):
            p = lax.rem(c, RB)
            q = lax.rem(c, RI)
            for g, (off, j0) in enumerate([(0, 0), (16, 0), (24, 8)]):
                di = idx_v[q, 0, pl.ds(off, LANES)]
                si = idx_v[q, 1, pl.ds(off, LANES)]
                t = (plsc.load_gather(s1_v, [si])
                     + plsc.load_gather(s2_v, [di]))
                t = jnp.where(t >= 0, t, ALPHA * t)
                h16 = jnp.exp(t)
                for j in range(j0, LANES):
                    hj = h16[j]
                    row = off + j
                    for kk in range(WPAD // LANES):
                        sl = pl.ds(kk * LANES, LANES)
                        rows_v[p, row, sl] = rows_v[p, row, sl] * hj

        def body(c, carry):
            wait_gather(c)
            work(c)
            issue_scatter(c)
            n2 = c + 2

            @pl.when(jnp.logical_and(n2 < CPT, c >= 1))
            def _():
                wait_scatter(c - 1)
                wait_idx(n2)
                issue_gather(n2)

                @pl.when(c + 5 < CPT)
                def _():
                    issue_idx(c + 5)

            @pl.when(jnp.logical_and(n2 < CPT, c < 1))
            def _():
                wait_idx(n2)
                issue_gather(n2)

                @pl.when(c + 5 < CPT)
                def _():
                    issue_idx(c + 5)

            return carry

        lax.fori_loop(0, CPT, body, 0)

        for cc in range(CPT - RB, CPT):
            wait_scatter(cc)
        plsc.subcore_barrier()

        pltpu.sync_copy(acc.at[pl.ds(sid * nstripe, nstripe)],
                        out_hbm.at[core].at[sid])

    return k(zpad, s1, s2, eidx)


def _tc_head_body(p_ref, idx_ref, g_ref, b_ref, W_ref, fb_ref, out_ref,
                  acc_ref):
    i = pl.program_id(0)

    @pl.when(i == 0)
    def _():
        acc_ref[...] = jnp.zeros((G, D), jnp.float32)
        out_ref[...] = jnp.zeros((G, C), jnp.float32)

    p = p_ref[0] + p_ref[1]
    den = p[:, D:D + 1]
    den = jnp.where(den > 0.0, den, 1.0)
    x = _leaky(p[:, :D] / den)
    ids = idx_ref[0, 0, :]
    gid = lax.broadcasted_iota(jnp.int32, (G, BLK), 0)
    onehot = jnp.where(gid == ids[None, :], 1.0, 0.0)
    acc_ref[...] = acc_ref[...] + jnp.dot(onehot, x,
                                          preferred_element_type=jnp.float32)

    @pl.when(i == NBLK - 1)
    def _():
        pooled = acc_ref[...]
        mean = jnp.mean(pooled, axis=0, keepdims=True)
        var = jnp.mean((pooled - mean) ** 2, axis=0, keepdims=True)
        xn = (pooled - mean) * lax.rsqrt(var + BN_EPS)
        xn = xn * g_ref[...].reshape(1, D) + b_ref[...].reshape(1, D)
        logits = lax.dot_general(xn, W_ref[...], (((1,), (1,)), ((), ())),
                                 preferred_element_type=jnp.float32)
        logits = logits + fb_ref[...].reshape(1, C)
        m = jnp.max(logits, axis=1, keepdims=True)
        s = jnp.sum(jnp.exp(logits - m), axis=1, keepdims=True)
        out_ref[...] = logits - (jnp.log(s) + m)


def _tc_head(p, idx3d, bn_gamma, bn_beta, fc_W, fc_b):
    return pl.pallas_call(
        _tc_head_body,
        grid=(NBLK,),
        in_specs=[
            pl.BlockSpec((2, BLK, WPAD), lambda i: (0, i, 0)),
            pl.BlockSpec((1, 1, BLK), lambda i: (i, 0, 0)),
            pl.BlockSpec((D,), lambda i: (0,)),
            pl.BlockSpec((D,), lambda i: (0,)),
            pl.BlockSpec((C, D), lambda i: (0, 0)),
            pl.BlockSpec((C,), lambda i: (0,)),
        ],
        out_specs=pl.BlockSpec((G, C), lambda i: (0, 0)),
        out_shape=jax.ShapeDtypeStruct((G, C), jnp.float32),
        scratch_shapes=[pltpu.VMEM((G, D), jnp.float32)],
    )(p, idx3d, bn_gamma, bn_beta, fc_W, fc_b)


def kernel(x_in, adj, idx, W1, a1, W2, a2, W3, a3, fc_W, fc_b,
           bn_gamma, bn_beta):
    src3d = adj[0].astype(jnp.int32).reshape(NTILE, CPT, CHUNK)
    dst3d = adj[1].astype(jnp.int32).reshape(NTILE, CPT, CHUNK)
    eidx = jnp.stack([dst3d, src3d], axis=2)
    idx3d = idx.astype(jnp.int32).reshape(NBLK, 1, BLK)

    z, s1, s2 = _tc_layer(x_in, W1, a1, is_first=True)
    p = _sc_edge(z, s1, s2, eidx).reshape(NC, N, WPAD)
    z, s1, s2 = _tc_layer(p, W2, a2, is_first=False)
    p = _sc_edge(z, s1, s2, eidx).reshape(NC, N, WPAD)
    z, s1, s2 = _tc_layer(p, W3, a3, is_first=False)
    p = _sc_edge(z, s1, s2, eidx).reshape(NC, N, WPAD)
    return _tc_head(p, idx3d, bn_gamma, bn_beta, fc_W, fc_b)
